# baseline (speedup 1.0000x reference)
import functools

import jax
import jax.numpy as jnp
from jax import lax
from jax.experimental import pallas as pl
from jax.experimental.pallas import tpu as pltpu
from jax.experimental.pallas import tpu_sc as plsc

V, L, DIM = 4096, 50, 64
B = V * L
NC, NS = 2, 16
NW = NC * NS
ROWS_W = B // NW
CHUNK = 128
NCHUNK = ROWS_W // CHUNK
NBUF = 5

@functools.lru_cache(maxsize=None)
def _get_sc_gather():
    mesh = plsc.VectorSubcoreMesh(
        core_axis_name="c", subcore_axis_name="s", num_cores=NC, num_subcores=NS
    )

    @functools.partial(
        pl.kernel,
        out_type=jax.ShapeDtypeStruct((B, DIM), jnp.float32),
        mesh=mesh,
        scratch_types=[
            pltpu.VMEM((NCHUNK, CHUNK), jnp.int32),
            [pltpu.VMEM((CHUNK, DIM), jnp.float32) for _ in range(NBUF)],
            [pltpu.SemaphoreType.DMA for _ in range(NBUF)],
        ],
        compiler_params=pltpu.CompilerParams(use_tc_tiling_on_sc=False),
    )
    def _sc_gather(idx_hbm, table_hbm, out_hbm, idx_v, bufs, sems):
        wid = lax.axis_index("s") * NC + lax.axis_index("c")
        base = wid * ROWS_W
        pltpu.sync_copy(idx_hbm.at[wid], idx_v)

        def start(c, b):
            pltpu.make_async_copy(
                table_hbm.at[idx_v.at[c]], bufs[b], sems[b]
            ).start()

        def wait(c, b):
            pltpu.make_async_copy(
                table_hbm.at[idx_v.at[c]], bufs[b], sems[b]
            ).wait()

        for b in range(NBUF):
            start(b, b)

        @pl.loop(0, NCHUNK, step=NBUF)
        def _(c0):
            for b in range(NBUF):
                c = c0 + b
                wait(c, b)
                pltpu.sync_copy(
                    bufs[b], out_hbm.at[pl.ds(base + c * CHUNK, CHUNK)]
                )

                @pl.when(c + NBUF < NCHUNK)
                def _():
                    start(c + NBUF, b)

    return _sc_gather


def _silu(x):
    return x * jax.nn.sigmoid(x)


BLK = 512


def _tc_body(fv_ref, x_ref, W1_ref, b1_ref, W2_ref, b2_ref, A1_ref, a1_ref,
             A2t_ref, a2_ref, R1_ref, r1_ref, R2_ref, r2_ref, out_ref):
    W1 = W1_ref[...]
    b1 = b1_ref[...][None, :]
    W2 = W2_ref[...]
    b2 = b2_ref[...][None, :]
    A1 = A1_ref[...]
    a1 = a1_ref[...][None, :]
    A2t = A2t_ref[...]
    a2 = a2_ref[...]
    R1 = R1_ref[...]
    r1 = r1_ref[...][None, :]
    R2 = R2_ref[...]
    r2 = r2_ref[...][None, :]

    def step(l, carry):
        m, s, acc = carry
        x = x_ref[l]
        h = _silu(jnp.dot(x, W1) + b1)
        h = _silu(jnp.dot(h, W2) + b2)
        t = jnp.tanh(jnp.dot(h, A1) + a1)
        logit = jnp.sum(t * A2t, axis=1, keepdims=True) + a2
        mask = fv_ref[l] != 0
        logit = jnp.where(mask, logit, jnp.float32(-1e30))
        m_new = jnp.maximum(m, logit)
        corr = jnp.exp(m - m_new)
        w = jnp.exp(logit - m_new)
        return m_new, s * corr + w, acc * corr + w * h

    m0 = jnp.full((BLK, 1), -1e30, jnp.float32)
    s0 = jnp.zeros((BLK, 1), jnp.float32)
    acc0 = jnp.zeros((BLK, DIM), jnp.float32)
    m, s, acc = lax.fori_loop(0, L, step, (m0, s0, acc0))
    h_pool = acc / s
    v = _silu(jnp.dot(h_pool, R1) + r1)
    out_ref[...] = jnp.dot(v, R2) + r2


_full = lambda *shape: pl.BlockSpec(shape, lambda i: (0,) * len(shape))

_tc_encode = pl.pallas_call(
    _tc_body,
    grid=(V // BLK,),
    in_specs=[
        pl.BlockSpec((L, BLK, 1), lambda i: (0, i, 0)),
        pl.BlockSpec((L, BLK, DIM), lambda i: (0, i, 0)),
        _full(DIM, DIM),
        _full(DIM),
        _full(DIM, DIM),
        _full(DIM),
        _full(DIM, DIM),
        _full(DIM),
        _full(1, DIM),
        _full(1, 1),
        _full(DIM, DIM),
        _full(DIM),
        _full(DIM, DIM),
        _full(DIM),
    ],
    out_specs=pl.BlockSpec((BLK, DIM), lambda i: (i, 0)),
    out_shape=jax.ShapeDtypeStruct((V, DIM), jnp.float32),
)


def kernel(flat_visits, emb, W1, b1, W2, b2, A1, a1, A2, a2, R1, r1, R2, r2):
    fv_t = flat_visits.T
    idx = fv_t.reshape(NW, NCHUNK, CHUNK)
    gx = _get_sc_gather()(idx, emb)
    x = gx.reshape(L, V, DIM)
    return _tc_encode(fv_t[:, :, None], x, W1, b1, W2, b2, A1, a1,
                      A2.T, a2.reshape(1, 1), R1, r1, R2, r2)


# speedup vs baseline: 1.1132x; 1.1132x over previous
import functools

import jax
import jax.numpy as jnp
from jax import lax
from jax.experimental import pallas as pl
from jax.experimental.pallas import tpu as pltpu
from jax.experimental.pallas import tpu_sc as plsc

V, L, DIM = 4096, 50, 64
PAD = 128
FLAGS = 16
B = V * L
NC, NS = 2, 16
NW = NC * NS
ROWS_W = B // NW
CHUNK = 128
NCHUNK = ROWS_W // CHUNK
NBUF = 5
SCL = 16


@functools.lru_cache(maxsize=None)
def _get_sc_gather():
    mesh = plsc.VectorSubcoreMesh(
        core_axis_name="c", subcore_axis_name="s", num_cores=NC, num_subcores=NS
    )

    @functools.partial(
        pl.kernel,
        out_type=jax.ShapeDtypeStruct((B, PAD), jnp.float32),
        mesh=mesh,
        scratch_types=[
            pltpu.VMEM((NCHUNK, CHUNK), jnp.int32),
            pltpu.VMEM((CHUNK, FLAGS), jnp.float32),
            [pltpu.VMEM((CHUNK, DIM), jnp.float32) for _ in range(NBUF)],
            [pltpu.SemaphoreType.DMA for _ in range(NBUF)],
        ],
        compiler_params=pltpu.CompilerParams(
            use_tc_tiling_on_sc=False, needs_layout_passes=False
        ),
    )
    def _sc_gather(idx_hbm, table_hbm, out_hbm, idx_v, fbuf, bufs, sems):
        wid = lax.axis_index("s") * NC + lax.axis_index("c")
        base = wid * ROWS_W
        pltpu.sync_copy(idx_hbm.at[pl.ds(wid * NCHUNK, NCHUNK)], idx_v)

        def start(c, b):
            pltpu.make_async_copy(
                table_hbm.at[idx_v.at[c]], bufs[b], sems[b]
            ).start()

        def wait(c, b):
            pltpu.make_async_copy(
                table_hbm.at[idx_v.at[c]], bufs[b], sems[b]
            ).wait()

        for b in range(NBUF):
            start(b, b)

        lane = lax.iota(jnp.int32, SCL)
        zeros = jnp.zeros((SCL,), jnp.int32)

        @pl.loop(0, NCHUNK, step=NBUF)
        def _(c0):
            for b in range(NBUF):
                c = c0 + b
                for g in range(CHUNK // SCL):
                    iv = idx_v[c, pl.ds(g * SCL, SCL)]
                    fl = jnp.where(iv != 0, 1.0, 0.0).astype(jnp.float32)
                    plsc.store_scatter(fbuf, [lane + g * SCL, zeros], fl)
                wait(c, b)
                rows = pl.ds(base + c * CHUNK, CHUNK)
                pltpu.sync_copy(bufs[b], out_hbm.at[rows, pl.ds(0, DIM)])
                pltpu.sync_copy(fbuf, out_hbm.at[rows, pl.ds(DIM, FLAGS)])

                @pl.when(c + NBUF < NCHUNK)
                def _():
                    start(c + NBUF, b)

    return _sc_gather


def _silu(x):
    return x * jax.nn.sigmoid(x)


BLK = 256


def _tc_body(x_ref, W1_ref, b1_ref, W2_ref, b2_ref, A1_ref, a1_ref,
             A2t_ref, a2_ref, R1_ref, r1_ref, R2_ref, r2_ref, out_ref):
    W1 = W1_ref[...]
    b1 = b1_ref[...][None, :]
    W2 = W2_ref[...]
    b2 = b2_ref[...][None, :]
    A1 = A1_ref[...]
    a1 = a1_ref[...][None, :]
    A2t = A2t_ref[...]
    a2 = a2_ref[...]
    R1 = R1_ref[...]
    r1 = r1_ref[...][None, :]
    R2 = R2_ref[...]
    r2 = r2_ref[...][None, :]

    xf = x_ref[...]
    x = xf[:, :, :DIM].reshape(L * BLK, DIM)
    flag = xf[:, :, DIM:DIM + 1].reshape(L * BLK, 1)
    h = _silu(jnp.dot(x, W1) + b1)
    h = _silu(jnp.dot(h, W2) + b2)
    t = jnp.tanh(jnp.dot(h, A1) + a1)
    logit = jnp.sum(t * A2t, axis=1, keepdims=True) + a2
    logit = jnp.where(flag != 0.0, logit, jnp.float32(-1e30))

    logit3 = logit.reshape(L, BLK, 1)
    m = logit3[0]
    for l in range(1, L):
        m = jnp.maximum(m, logit3[l])
    w3 = jnp.exp(logit3 - m[None])
    s = jnp.sum(w3, axis=0)
    pooled = jnp.sum(w3 * h.reshape(L, BLK, DIM), axis=0)

    h_pool = pooled / s
    v = _silu(jnp.dot(h_pool, R1) + r1)
    out_ref[...] = jnp.dot(v, R2) + r2


_full = lambda *shape: pl.BlockSpec(shape, lambda i: (0,) * len(shape))

_tc_encode = pl.pallas_call(
    _tc_body,
    grid=(V // BLK,),
    in_specs=[
        pl.BlockSpec((L, BLK, PAD), lambda i: (0, i, 0)),
        _full(DIM, DIM),
        _full(DIM),
        _full(DIM, DIM),
        _full(DIM),
        _full(DIM, DIM),
        _full(DIM),
        _full(1, DIM),
        _full(1, 1),
        _full(DIM, DIM),
        _full(DIM),
        _full(DIM, DIM),
        _full(DIM),
    ],
    out_specs=pl.BlockSpec((BLK, DIM), lambda i: (i, 0)),
    out_shape=jax.ShapeDtypeStruct((V, DIM), jnp.float32),
)


def kernel(flat_visits, emb, W1, b1, W2, b2, A1, a1, A2, a2, R1, r1, R2, r2):
    fv_t = flat_visits.T
    idx = fv_t.reshape(B // CHUNK, CHUNK)
    gx = _get_sc_gather()(idx, emb)
    x = gx.reshape(L, V, PAD)
    return _tc_encode(x, W1, b1, W2, b2, A1, a1,
                      A2.T, a2.reshape(1, 1), R1, r1, R2, r2)


# speedup vs baseline: 2.6897x; 2.4162x over previous
import functools

import jax
import jax.numpy as jnp
from jax import lax
from jax.experimental import pallas as pl
from jax.experimental.pallas import tpu as pltpu
from jax.experimental.pallas import tpu_sc as plsc

V, L, DIM = 4096, 50, 64
LP = L // 2
PAD = 128
B = V * L
NC, NS = 2, 16
NW = NC * NS
ROWS_W = B // NW
CHUNK = 128
HALF = CHUNK // 2
NCHUNK = ROWS_W // CHUNK
NBUF = 5


@functools.lru_cache(maxsize=None)
def _get_sc_gather():
    mesh = plsc.VectorSubcoreMesh(
        core_axis_name="c", subcore_axis_name="s", num_cores=NC, num_subcores=NS
    )

    @functools.partial(
        pl.kernel,
        out_type=jax.ShapeDtypeStruct((B // 2, PAD), jnp.float32),
        mesh=mesh,
        scratch_types=[
            pltpu.VMEM((NCHUNK, CHUNK), jnp.int32),
            [pltpu.VMEM((CHUNK, DIM), jnp.float32) for _ in range(NBUF)],
            [pltpu.SemaphoreType.DMA for _ in range(NBUF)],
        ],
        compiler_params=pltpu.CompilerParams(
            use_tc_tiling_on_sc=False, needs_layout_passes=False
        ),
    )
    def _sc_gather(idx_hbm, table_hbm, out_hbm, idx_v, bufs, sems):
        wid = lax.axis_index("s") * NC + lax.axis_index("c")
        base = wid * ROWS_W
        pltpu.sync_copy(idx_hbm.at[pl.ds(wid * NCHUNK, NCHUNK)], idx_v)

        def start(c, b):
            pltpu.make_async_copy(
                table_hbm.at[idx_v.at[c]], bufs[b], sems[b]
            ).start()

        def wait(c, b):
            pltpu.make_async_copy(
                table_hbm.at[idx_v.at[c]], bufs[b], sems[b]
            ).wait()

        for b in range(NBUF):
            start(b, b)

        @pl.loop(0, NCHUNK, step=NBUF)
        def _(c0):
            for b in range(NBUF):
                c = c0 + b
                wait(c, b)
                orow = pl.ds((base + c * CHUNK) // 2, HALF)
                pltpu.sync_copy(bufs[b].at[pl.ds(0, HALF)],
                                out_hbm.at[orow, pl.ds(0, DIM)])
                pltpu.sync_copy(bufs[b].at[pl.ds(HALF, HALF)],
                                out_hbm.at[orow, pl.ds(DIM, DIM)])

                @pl.when(c + NBUF < NCHUNK)
                def _():
                    start(c + NBUF, b)

    return _sc_gather


VOCAB = 1000000
WCW = 8192


def _widen_body(xt_ref, out_ref):
    xt = xt_ref[...]
    ii = lax.broadcasted_iota(jnp.int32, (DIM, DIM), 0)
    jj = lax.broadcasted_iota(jnp.int32, (DIM, DIM), 1)
    eye = (ii == jj).astype(jnp.float32)
    rows = lax.dot_general(xt, eye, (((0,), (0,)), ((), ())))
    out_ref[...] = jnp.concatenate(
        [rows, jnp.zeros((WCW, PAD - DIM), jnp.float32)], axis=1)


_widen = pl.pallas_call(
    _widen_body,
    grid=((VOCAB + WCW - 1) // WCW,),
    in_specs=[pl.BlockSpec((DIM, WCW), lambda i: (0, i))],
    out_specs=pl.BlockSpec((WCW, PAD), lambda i: (i, 0)),
    out_shape=jax.ShapeDtypeStruct((VOCAB, PAD), jnp.float32),
)


def _silu(x):
    half = 0.5 * x
    return half * jnp.tanh(half) + half


BLK = 256


def _tc_body(fv_ref, x_ref, W1_ref, b1_ref, W2_ref, b2_ref, A1_ref, a1_ref,
             A2_ref, a2_ref, R1_ref, r1_ref, R2_ref, r2_ref, out_ref):
    W1 = W1_ref[...]
    b1 = b1_ref[...][None, :]
    W2 = W2_ref[...]
    b2 = b2_ref[...][None, :]
    A1 = A1_ref[...]
    a1 = a1_ref[...][None, :]
    A2 = A2_ref[...]
    a2 = a2_ref[...]
    R1 = R1_ref[...]
    r1 = r1_ref[...][None, :]
    R2 = R2_ref[...]
    r2 = r2_ref[...][None, :]
    fvb = fv_ref[...]

    x = x_ref[...].reshape(LP * BLK, PAD)
    h = _silu(jnp.dot(x, W1) + b1)
    h = _silu(jnp.dot(h, W2) + b2)
    t = jnp.tanh(jnp.dot(h, A1) + a1)
    logit = jnp.dot(t, A2) + a2
    logit3 = logit.reshape(LP, BLK, 2)

    masked = [
        jnp.where(fvb[:, 2 * l2:2 * l2 + 2] != 0, logit3[l2],
                  jnp.float32(-1e30))
        for l2 in range(LP)
    ]
    m = masked[0]
    for l2 in range(1, LP):
        m = jnp.maximum(m, masked[l2])
    m = jnp.max(m, axis=1, keepdims=True)
    w = jnp.exp(jnp.stack(masked, axis=0) - m[None])
    s = jnp.sum(jnp.sum(w, axis=0), axis=1, keepdims=True)

    li = lax.broadcasted_iota(jnp.int32, (2, PAD), 1) // DIM
    ri = lax.broadcasted_iota(jnp.int32, (2, PAD), 0)
    sel = (li == ri).astype(jnp.float32)
    w128 = jnp.dot(w.reshape(LP * BLK, 2), sel)
    pooled2 = jnp.sum(w128.reshape(LP, BLK, PAD) * h.reshape(LP, BLK, PAD),
                      axis=0)
    pooled = pooled2[:, :DIM] + pooled2[:, DIM:]

    h_pool = pooled / s
    v = _silu(jnp.dot(h_pool, R1) + r1)
    out_ref[...] = jnp.dot(v, R2) + r2


_full = lambda *shape: pl.BlockSpec(shape, lambda i: (0,) * len(shape))

_tc_encode = pl.pallas_call(
    _tc_body,
    grid=(V // BLK,),
    in_specs=[
        pl.BlockSpec((BLK, L), lambda i: (i, 0)),
        pl.BlockSpec((LP, BLK, PAD), lambda i: (0, i, 0)),
        _full(PAD, PAD),
        _full(PAD),
        _full(PAD, PAD),
        _full(PAD),
        _full(PAD, PAD),
        _full(PAD),
        _full(PAD, 2),
        _full(1, 1),
        _full(DIM, DIM),
        _full(DIM),
        _full(DIM, DIM),
        _full(DIM),
    ],
    out_specs=pl.BlockSpec((BLK, DIM), lambda i: (i, 0)),
    out_shape=jax.ShapeDtypeStruct((V, DIM), jnp.float32),
)


def _pair_chunk_indices(flat_visits):
    fv_t = flat_visits.T
    a = fv_t.reshape(LP, 2, V).transpose(0, 2, 1)
    a = a.reshape(B // CHUNK, HALF, 2).transpose(0, 2, 1)
    return a.reshape(B // CHUNK, CHUNK) * 2


def kernel(flat_visits, emb, W1, b1, W2, b2, A1, a1, A2, a2, R1, r1, R2, r2):
    table = _widen(emb.T).reshape(2 * VOCAB, DIM)
    idx = _pair_chunk_indices(flat_visits)
    gx = _get_sc_gather()(idx, table)
    x = gx.reshape(LP, V, PAD)
    eye2 = jnp.eye(2, dtype=jnp.float32)
    W1b = jnp.kron(eye2, W1)
    W2b = jnp.kron(eye2, W2)
    A1b = jnp.kron(eye2, A1)
    A2b = jnp.kron(eye2, A2)
    return _tc_encode(flat_visits, x, W1b, jnp.tile(b1, 2), W2b,
                      jnp.tile(b2, 2), A1b, jnp.tile(a1, 2), A2b,
                      a2.reshape(1, 1), R1, r1, R2, r2)


# speedup vs baseline: 3.1033x; 1.1538x over previous
import functools

import jax
import jax.numpy as jnp
from jax import lax
from jax.experimental import pallas as pl
from jax.experimental.pallas import tpu as pltpu
from jax.experimental.pallas import tpu_sc as plsc

V, L, DIM = 4096, 50, 64
LP = L // 2
PAD = 128
B = V * L
NC, NS = 2, 16
NW = NC * NS
ROWS_W = B // NW
CHUNK = 128
HALF = CHUNK // 2
NCHUNK = ROWS_W // CHUNK
NBUF = 5


@functools.lru_cache(maxsize=None)
def _get_sc_gather():
    mesh = plsc.VectorSubcoreMesh(
        core_axis_name="c", subcore_axis_name="s", num_cores=NC, num_subcores=NS
    )

    @functools.partial(
        pl.kernel,
        out_type=jax.ShapeDtypeStruct((B // 2, PAD), jnp.float32),
        mesh=mesh,
        scratch_types=[
            pltpu.VMEM((NCHUNK, CHUNK), jnp.int32),
            [pltpu.VMEM((CHUNK, DIM), jnp.float32) for _ in range(NBUF)],
            [pltpu.SemaphoreType.DMA for _ in range(NBUF)],
        ],
        compiler_params=pltpu.CompilerParams(
            use_tc_tiling_on_sc=False, needs_layout_passes=False
        ),
    )
    def _sc_gather(idx_hbm, table_hbm, out_hbm, idx_v, bufs, sems):
        wid = lax.axis_index("s") * NC + lax.axis_index("c")
        base = wid * ROWS_W
        pltpu.sync_copy(idx_hbm.at[pl.ds(wid * NCHUNK, NCHUNK)], idx_v)

        def start(c, b):
            pltpu.make_async_copy(
                table_hbm.at[idx_v.at[c]], bufs[b], sems[b]
            ).start()

        def wait(c, b):
            pltpu.make_async_copy(
                table_hbm.at[idx_v.at[c]], bufs[b], sems[b]
            ).wait()

        for b in range(NBUF):
            start(b, b)

        @pl.loop(0, NCHUNK, step=NBUF)
        def _(c0):
            for b in range(NBUF):
                c = c0 + b
                wait(c, b)
                orow = pl.ds((base + c * CHUNK) // 2, HALF)
                pltpu.sync_copy(bufs[b].at[pl.ds(0, HALF)],
                                out_hbm.at[orow, pl.ds(0, DIM)])
                pltpu.sync_copy(bufs[b].at[pl.ds(HALF, HALF)],
                                out_hbm.at[orow, pl.ds(DIM, DIM)])

                @pl.when(c + NBUF < NCHUNK)
                def _():
                    start(c + NBUF, b)

    return _sc_gather


VOCAB = 1000000
SPLIT = 512000
WCW = 16000


def _widen_body(a_ref, b_ref, out_ref):
    ii = lax.broadcasted_iota(jnp.int32, (DIM, DIM), 0)
    jj = lax.broadcasted_iota(jnp.int32, (DIM, DIM), 1)
    eye = (ii == jj).astype(jnp.float32)
    tr = lambda x: lax.dot_general(x, eye, (((0,), (0,)), ((), ())))
    out_ref[...] = jnp.concatenate([tr(a_ref[...]), tr(b_ref[...])], axis=1)


_widen = pl.pallas_call(
    _widen_body,
    grid=(SPLIT // WCW,),
    in_specs=[
        pl.BlockSpec((DIM, WCW), lambda i: (0, i)),
        pl.BlockSpec(
            (DIM, WCW),
            lambda i: (0, jnp.minimum(i + SPLIT // WCW, VOCAB // WCW)),
        ),
    ],
    out_specs=pl.BlockSpec((WCW, PAD), lambda i: (i, 0)),
    out_shape=jax.ShapeDtypeStruct((SPLIT, PAD), jnp.float32),
)


def _silu(x):
    half = 0.5 * x
    return half * jnp.tanh(half) + half


BLK = 256


def _tc_body(fv_ref, x_ref, W1_ref, b1_ref, W2_ref, b2_ref, A1_ref, a1_ref,
             A2_ref, a2_ref, R1_ref, r1_ref, R2_ref, r2_ref, out_ref):
    W1 = W1_ref[...]
    b1 = b1_ref[...][None, :]
    W2 = W2_ref[...]
    b2 = b2_ref[...][None, :]
    A1 = A1_ref[...]
    a1 = a1_ref[...][None, :]
    A2 = A2_ref[...]
    a2 = a2_ref[...]
    R1 = R1_ref[...]
    r1 = r1_ref[...][None, :]
    R2 = R2_ref[...]
    r2 = r2_ref[...][None, :]
    fvb = fv_ref[...]

    x = x_ref[...].reshape(LP * BLK, PAD)
    h = _silu(jnp.dot(x, W1) + b1)
    h = _silu(jnp.dot(h, W2) + b2)
    t = jnp.tanh(jnp.dot(h, A1) + a1)
    logit = jnp.dot(t, A2) + a2
    logit3 = logit.reshape(LP, BLK, 2)

    masked = [
        jnp.where(fvb[:, 2 * l2:2 * l2 + 2] != 0, logit3[l2],
                  jnp.float32(-1e30))
        for l2 in range(LP)
    ]
    m = masked[0]
    for l2 in range(1, LP):
        m = jnp.maximum(m, masked[l2])
    m = jnp.max(m, axis=1, keepdims=True)
    w = jnp.exp(jnp.stack(masked, axis=0) - m[None])
    s = jnp.sum(jnp.sum(w, axis=0), axis=1, keepdims=True)

    li = lax.broadcasted_iota(jnp.int32, (2, PAD), 1) // DIM
    ri = lax.broadcasted_iota(jnp.int32, (2, PAD), 0)
    sel = (li == ri).astype(jnp.float32)
    w128 = jnp.dot(w.reshape(LP * BLK, 2), sel)
    pooled2 = jnp.sum(w128.reshape(LP, BLK, PAD) * h.reshape(LP, BLK, PAD),
                      axis=0)
    pooled = pooled2[:, :DIM] + pooled2[:, DIM:]

    h_pool = pooled / s
    v = _silu(jnp.dot(h_pool, R1) + r1)
    out_ref[...] = jnp.dot(v, R2) + r2


_full = lambda *shape: pl.BlockSpec(shape, lambda i: (0,) * len(shape))

_tc_encode = pl.pallas_call(
    _tc_body,
    grid=(V // BLK,),
    in_specs=[
        pl.BlockSpec((BLK, L), lambda i: (i, 0)),
        pl.BlockSpec((LP, BLK, PAD), lambda i: (0, i, 0)),
        _full(PAD, PAD),
        _full(PAD),
        _full(PAD, PAD),
        _full(PAD),
        _full(PAD, PAD),
        _full(PAD),
        _full(PAD, 2),
        _full(1, 1),
        _full(DIM, DIM),
        _full(DIM),
        _full(DIM, DIM),
        _full(DIM),
    ],
    out_specs=pl.BlockSpec((BLK, DIM), lambda i: (i, 0)),
    out_shape=jax.ShapeDtypeStruct((V, DIM), jnp.float32),
)


def _pair_chunk_indices(flat_visits):
    fv_t = flat_visits.T
    a = fv_t.reshape(LP, 2, V).transpose(0, 2, 1)
    a = a.reshape(B // CHUNK, HALF, 2).transpose(0, 2, 1)
    v = a.reshape(B // CHUNK, CHUNK)
    return jnp.where(v < SPLIT, 2 * v, 2 * (v - SPLIT) + 1)


def kernel(flat_visits, emb, W1, b1, W2, b2, A1, a1, A2, a2, R1, r1, R2, r2):
    embT = emb.T
    table = _widen(embT, embT).reshape(2 * SPLIT, DIM)
    idx = _pair_chunk_indices(flat_visits)
    gx = _get_sc_gather()(idx, table)
    x = gx.reshape(LP, V, PAD)
    eye2 = jnp.eye(2, dtype=jnp.float32)
    W1b = jnp.kron(eye2, W1)
    W2b = jnp.kron(eye2, W2)
    A1b = jnp.kron(eye2, A1)
    A2b = jnp.kron(eye2, A2)
    return _tc_encode(flat_visits, x, W1b, jnp.tile(b1, 2), W2b,
                      jnp.tile(b2, 2), A1b, jnp.tile(a1, 2), A2b,
                      a2.reshape(1, 1), R1, r1, R2, r2)


# speedup vs baseline: 3.1641x; 1.0196x over previous
import functools

import jax
import jax.numpy as jnp
from jax import lax
from jax.experimental import pallas as pl
from jax.experimental.pallas import tpu as pltpu
from jax.experimental.pallas import tpu_sc as plsc

V, L, DIM = 4096, 50, 64
LP = L // 2
PAD = 128
B = V * L
NC, NS = 2, 16
NW = NC * NS
ROWS_W = B // NW
CHUNK = 128
HALF = CHUNK // 2
NCHUNK = ROWS_W // CHUNK
NBUF = 5


@functools.lru_cache(maxsize=None)
def _get_sc_gather():
    mesh = plsc.VectorSubcoreMesh(
        core_axis_name="c", subcore_axis_name="s", num_cores=NC, num_subcores=NS
    )

    @functools.partial(
        pl.kernel,
        out_type=jax.ShapeDtypeStruct((B // 2, PAD), jnp.float32),
        mesh=mesh,
        scratch_types=[
            pltpu.VMEM((NCHUNK, CHUNK), jnp.int32),
            [pltpu.VMEM((CHUNK, DIM), jnp.float32) for _ in range(NBUF)],
            [pltpu.SemaphoreType.DMA for _ in range(NBUF)],
        ],
        compiler_params=pltpu.CompilerParams(
            use_tc_tiling_on_sc=False, needs_layout_passes=False
        ),
    )
    def _sc_gather(idx_hbm, table_hbm, out_hbm, idx_v, bufs, sems):
        wid = lax.axis_index("s") * NC + lax.axis_index("c")
        base = wid * ROWS_W
        pltpu.sync_copy(idx_hbm.at[pl.ds(wid * NCHUNK, NCHUNK)], idx_v)

        def start(c, b):
            pltpu.make_async_copy(
                table_hbm.at[idx_v.at[c]], bufs[b], sems[b]
            ).start()

        def wait(c, b):
            pltpu.make_async_copy(
                table_hbm.at[idx_v.at[c]], bufs[b], sems[b]
            ).wait()

        for b in range(NBUF):
            start(b, b)

        @pl.loop(0, NCHUNK, step=NBUF)
        def _(c0):
            for b in range(NBUF):
                c = c0 + b
                wait(c, b)
                orow = pl.ds((base + c * CHUNK) // 2, HALF)
                pltpu.sync_copy(bufs[b].at[pl.ds(0, HALF)],
                                out_hbm.at[orow, pl.ds(0, DIM)])
                pltpu.sync_copy(bufs[b].at[pl.ds(HALF, HALF)],
                                out_hbm.at[orow, pl.ds(DIM, DIM)])

                @pl.when(c + NBUF < NCHUNK)
                def _():
                    start(c + NBUF, b)

    return _sc_gather


VOCAB = 1000000
SPLIT = 512000
WCW = 16000


def _widen_body(a_ref, b_ref, out_ref):
    ii = lax.broadcasted_iota(jnp.int32, (DIM, DIM), 0)
    jj = lax.broadcasted_iota(jnp.int32, (DIM, DIM), 1)
    eye = (ii == jj).astype(jnp.float32)
    tr = lambda x: lax.dot_general(x, eye, (((0,), (0,)), ((), ())))
    out_ref[...] = jnp.concatenate([tr(a_ref[...]), tr(b_ref[...])], axis=1)


_widen = pl.pallas_call(
    _widen_body,
    grid=(SPLIT // WCW,),
    in_specs=[
        pl.BlockSpec((DIM, WCW), lambda i: (0, i)),
        pl.BlockSpec(
            (DIM, WCW),
            lambda i: (0, jnp.minimum(i + SPLIT // WCW, VOCAB // WCW)),
        ),
    ],
    out_specs=pl.BlockSpec((WCW, PAD), lambda i: (i, 0)),
    out_shape=jax.ShapeDtypeStruct((SPLIT, PAD), jnp.float32),
)


def _silu(x):
    half = 0.5 * x
    return half * jnp.tanh(half) + half


BLK = 512


def _tc_body(fv_ref, x_ref, W1_ref, b1_ref, W2_ref, b2_ref, A1_ref, a1_ref,
             A2_ref, a2_ref, R1_ref, r1_ref, R2_ref, r2_ref, out_ref):
    W1 = W1_ref[...]
    b1 = b1_ref[...][None, :]
    W2 = W2_ref[...]
    b2 = b2_ref[...][None, :]
    A1 = A1_ref[...]
    a1 = a1_ref[...][None, :]
    A2 = A2_ref[...]
    a2 = a2_ref[...]
    R1 = R1_ref[...]
    r1 = r1_ref[...][None, :]
    R2 = R2_ref[...]
    r2 = r2_ref[...][None, :]
    fvb = fv_ref[...]

    x = x_ref[...].reshape(LP * BLK, PAD)
    h = _silu(jnp.dot(x, W1) + b1)
    h = _silu(jnp.dot(h, W2) + b2)
    t = jnp.tanh(jnp.dot(h, A1) + a1)
    logit = jnp.dot(t, A2) + a2
    logit3 = logit.reshape(LP, BLK, 2)

    masked = [
        jnp.where(fvb[:, 2 * l2:2 * l2 + 2] != 0, logit3[l2],
                  jnp.float32(-1e30))
        for l2 in range(LP)
    ]
    m = masked[0]
    for l2 in range(1, LP):
        m = jnp.maximum(m, masked[l2])
    m = jnp.max(m, axis=1, keepdims=True)
    w = jnp.exp(jnp.stack(masked, axis=0) - m[None])
    s = jnp.sum(jnp.sum(w, axis=0), axis=1, keepdims=True)

    li = lax.broadcasted_iota(jnp.int32, (2, PAD), 1) // DIM
    ri = lax.broadcasted_iota(jnp.int32, (2, PAD), 0)
    sel = (li == ri).astype(jnp.float32)
    w128 = jnp.dot(w.reshape(LP * BLK, 2), sel)
    pooled2 = jnp.sum(w128.reshape(LP, BLK, PAD) * h.reshape(LP, BLK, PAD),
                      axis=0)
    pooled = pooled2[:, :DIM] + pooled2[:, DIM:]

    h_pool = pooled / s
    v = _silu(jnp.dot(h_pool, R1) + r1)
    out_ref[...] = jnp.dot(v, R2) + r2


_full = lambda *shape: pl.BlockSpec(shape, lambda i: (0,) * len(shape))

_tc_encode = pl.pallas_call(
    _tc_body,
    grid=(V // BLK,),
    in_specs=[
        pl.BlockSpec((BLK, L), lambda i: (i, 0)),
        pl.BlockSpec((LP, BLK, PAD), lambda i: (0, i, 0)),
        _full(PAD, PAD),
        _full(PAD),
        _full(PAD, PAD),
        _full(PAD),
        _full(PAD, PAD),
        _full(PAD),
        _full(PAD, 2),
        _full(1, 1),
        _full(DIM, DIM),
        _full(DIM),
        _full(DIM, DIM),
        _full(DIM),
    ],
    out_specs=pl.BlockSpec((BLK, DIM), lambda i: (i, 0)),
    out_shape=jax.ShapeDtypeStruct((V, DIM), jnp.float32),
)


def _pair_chunk_indices(flat_visits):
    fv_t = flat_visits.T
    a = fv_t.reshape(LP, 2, V).transpose(0, 2, 1)
    a = a.reshape(B // CHUNK, HALF, 2).transpose(0, 2, 1)
    v = a.reshape(B // CHUNK, CHUNK)
    return jnp.where(v < SPLIT, 2 * v, 2 * (v - SPLIT) + 1)


def kernel(flat_visits, emb, W1, b1, W2, b2, A1, a1, A2, a2, R1, r1, R2, r2):
    embT = emb.T
    table = _widen(embT, embT).reshape(2 * SPLIT, DIM)
    idx = _pair_chunk_indices(flat_visits)
    gx = _get_sc_gather()(idx, table)
    x = gx.reshape(LP, V, PAD)
    eye2 = jnp.eye(2, dtype=jnp.float32)
    W1b = jnp.kron(eye2, W1)
    W2b = jnp.kron(eye2, W2)
    A1b = jnp.kron(eye2, A1)
    A2b = jnp.kron(eye2, A2)
    return _tc_encode(flat_visits, x, W1b, jnp.tile(b1, 2), W2b,
                      jnp.tile(b2, 2), A1b, jnp.tile(a1, 2), A2b,
                      a2.reshape(1, 1), R1, r1, R2, r2)
